# Initial kernel scaffold; baseline (speedup 1.0000x reference)
import jax
import jax.numpy as jnp
from jax.experimental import pallas as pl


def kernel(points, label, params):
    raise NotImplementedError("write your pallas kernel here")


# speedup vs baseline: 12.5190x; 12.5190x over previous
import functools

import jax
import jax.numpy as jnp
from jax import lax
from jax.experimental import pallas as pl
from jax.experimental.pallas import tpu as pltpu
from jax.experimental.pallas import tpu_sc as plsc

B = 4
N = 4096
K = 10
BN = B * N
ROWS = BN * K
EPS = 1e-5
_BF = jnp.bfloat16

_NC = 2
_NS = 16
_NW = _NC * _NS


def _leaky(x):
    return jax.nn.leaky_relu(x, 0.2)


def _bn_apply(z, ga, be, m, v):
    return ga * (z - m) / jnp.sqrt(v + EPS) + be


def _stats_update(st_ref, z, first):
    s1 = jnp.sum(z, axis=0)
    s2 = jnp.sum(z * z, axis=0)
    c = s1.shape[0]
    row = lax.broadcasted_iota(jnp.int32, (8, c), 0)
    upd = (jnp.where(row == 0, s1[None, :], 0.0)
           + jnp.where(row == 1, s2[None, :], 0.0))

    @pl.when(first)
    def _():
        st_ref[...] = jnp.zeros((8, c), jnp.float32)

    st_ref[...] += upd


def _finalize_stats(st, count):
    m = st[0] / count
    v = st[1] / count - m * m
    return m.reshape(1, -1), v.reshape(1, -1)


def _stats_tile(st_ref, z):
    s1 = jnp.sum(z, axis=0)
    c = s1.shape[0]
    row = lax.broadcasted_iota(jnp.int32, (8, c), 0)
    st_ref[...] = jnp.where(row == 0, s1[None, :], 0.0)[None]


def _var_pass(z, m, count):
    c = z.shape[-1]
    nt = BN // _TG

    def body(z_ref, m_ref, st_ref):
        diff = z_ref[...].reshape(_TG * K, c) - m_ref[0][None, :]
        _stats_tile(st_ref, diff * diff)

    st = pl.pallas_call(
        body,
        grid=(nt,),
        in_specs=[
            pl.BlockSpec((_TG, K, c), lambda t: (t, 0, 0)),
            pl.BlockSpec((1, c), lambda t: (0, 0)),
        ],
        out_specs=pl.BlockSpec((1, 8, c), lambda t: (t, 0, 0)),
        out_shape=jax.ShapeDtypeStruct((nt, 8, c), jnp.float32),
        compiler_params=pltpu.CompilerParams(
            dimension_semantics=("parallel",)),
    )(z, m)
    return (jnp.sum(st[:, 0, :], axis=0) * (1.0 / count)).reshape(1, -1)


def _knn(xn, xtb, d):
    tn = 256

    def body(xa_ref, xt_ref, xT_ref, out_ref):
        xa = xa_ref[0]
        xt = xt_ref[0]
        sq_all = jnp.sum(xa * xa, axis=1)
        sq_t = jnp.sum(xt * xt, axis=1)
        inner = lax.dot_general(
            xt.astype(_BF), xT_ref[0], (((1,), (0,)), ((), ())),
            preferred_element_type=jnp.float32)
        dist = (sq_t[:, None] + (-2.0 * inner)) + sq_all[None, :]
        lane = lax.broadcasted_iota(jnp.int32, (tn, N), 1)
        dcur = dist
        for j in range(K + 1):
            mval = jnp.min(dcur, axis=1)
            cand = jnp.where(dcur == mval[:, None], lane, jnp.int32(N))
            amin = jnp.min(cand, axis=1)
            if j > 0:
                out_ref[0, j - 1, :] = amin
            if j < K:
                dcur = jnp.where(lane == amin[:, None], jnp.inf, dcur)

    return pl.pallas_call(
        body,
        grid=(B, N // tn),
        in_specs=[
            pl.BlockSpec((1, N, d), lambda b, t: (b, 0, 0)),
            pl.BlockSpec((1, tn, d), lambda b, t: (b, t, 0)),
            pl.BlockSpec((1, d, N), lambda b, t: (b, 0, 0)),
        ],
        out_specs=pl.BlockSpec((1, 16, tn), lambda b, t: (b, 0, t)),
        out_shape=jax.ShapeDtypeStruct((B, 16, N), jnp.int32),
        compiler_params=pltpu.CompilerParams(
            dimension_semantics=("parallel", "parallel")),
    )(xn, xn, xtb)


def _sc_gather(table, idxg, d):
    per_w = ROWS // _NW
    chunk = 512
    n_chunks = per_w // chunk
    mesh = plsc.VectorSubcoreMesh(core_axis_name="c", subcore_axis_name="s")

    @functools.partial(
        pl.kernel,
        mesh=mesh,
        out_type=jax.ShapeDtypeStruct((ROWS, d), jnp.float32),
        compiler_params=pltpu.CompilerParams(use_tc_tiling_on_sc=False),
        scratch_types=[
            pltpu.VMEM((chunk,), jnp.int32),
            pltpu.VMEM((chunk, d), jnp.float32),
            pltpu.SemaphoreType.DMA,
        ],
    )
    def gk(tab_hbm, idx_hbm, out_hbm, idx_v, rows_v, sem):
        wid = lax.axis_index("s") * _NC + lax.axis_index("c")
        base = wid * per_w

        def step(c, carry):
            off = base + c * chunk
            pltpu.sync_copy(idx_hbm.at[pl.ds(off, chunk)], idx_v)
            pltpu.async_copy(tab_hbm.at[idx_v], rows_v, sem).wait()
            pltpu.sync_copy(rows_v, out_hbm.at[pl.ds(off, chunk)])
            return carry

        lax.fori_loop(0, n_chunks, step, 0)

    return gk(table, idxg)


_TG = 512


def _conv0(gath, xflat, xsflat, w2d, d, cout):
    packed = xsflat is not None
    k2 = d if packed else 2 * d

    def body(g_ref, x_ref, *rest):
        if packed:
            xs_ref, w_ref, z_ref, st_ref = rest
        else:
            w_ref, z_ref, st_ref = rest
        x = x_ref[...]
        g = g_ref[...]
        diff = g - x[:, None, :]
        if packed:
            f = diff + xs_ref[...][:, None, :]
        else:
            f = jnp.concatenate(
                [diff, jnp.broadcast_to(x[:, None, :], g.shape)], axis=2)
        fb = f.astype(_BF).reshape(_TG * K, k2)
        z = lax.dot_general(fb, w_ref[...], (((1,), (0,)), ((), ())),
                            preferred_element_type=jnp.float32)
        z_ref[...] = z.reshape(_TG, K, cout)
        _stats_tile(st_ref, z)

    in_specs = [
        pl.BlockSpec((_TG, K, d), lambda t: (t, 0, 0)),
        pl.BlockSpec((_TG, d), lambda t: (t, 0)),
    ]
    args = [gath, xflat]
    if packed:
        in_specs.append(pl.BlockSpec((_TG, d), lambda t: (t, 0)))
        args.append(xsflat)
    in_specs.append(pl.BlockSpec((k2, cout), lambda t: (0, 0)))
    args.append(w2d)
    nt = BN // _TG
    return pl.pallas_call(
        body,
        grid=(nt,),
        in_specs=in_specs,
        out_specs=[
            pl.BlockSpec((_TG, K, cout), lambda t: (t, 0, 0)),
            pl.BlockSpec((1, 8, cout), lambda t: (t, 0, 0)),
        ],
        out_shape=[
            jax.ShapeDtypeStruct((BN, K, cout), jnp.float32),
            jax.ShapeDtypeStruct((nt, 8, cout), jnp.float32),
        ],
        compiler_params=pltpu.CompilerParams(
            dimension_semantics=("parallel",)),
    )(*args)


def _conv_mid(z0, ga, be, m, v, wt, cout):
    cin = z0.shape[-1]

    def body(z0_ref, ga_ref, be_ref, m_ref, v_ref, w_ref, z1_ref, st_ref):
        z0v = z0_ref[...]
        f = _leaky(_bn_apply(z0v, ga_ref[0], be_ref[0], m_ref[0], v_ref[0]))
        fb = f.astype(_BF).reshape(_TG * K, cin)
        z1 = lax.dot_general(fb, w_ref[...], (((1,), (0,)), ((), ())),
                             preferred_element_type=jnp.float32)
        z1_ref[...] = z1.reshape(_TG, K, cout)
        _stats_tile(st_ref, z1)

    nt = BN // _TG
    return pl.pallas_call(
        body,
        grid=(nt,),
        in_specs=[
            pl.BlockSpec((_TG, K, cin), lambda t: (t, 0, 0)),
            pl.BlockSpec((1, cin), lambda t: (0, 0)),
            pl.BlockSpec((1, cin), lambda t: (0, 0)),
            pl.BlockSpec((1, cin), lambda t: (0, 0)),
            pl.BlockSpec((1, cin), lambda t: (0, 0)),
            pl.BlockSpec((cin, cout), lambda t: (0, 0)),
        ],
        out_specs=[
            pl.BlockSpec((_TG, K, cout), lambda t: (t, 0, 0)),
            pl.BlockSpec((1, 8, cout), lambda t: (t, 0, 0)),
        ],
        out_shape=[
            jax.ShapeDtypeStruct((BN, K, cout), jnp.float32),
            jax.ShapeDtypeStruct((nt, 8, cout), jnp.float32),
        ],
        compiler_params=pltpu.CompilerParams(
            dimension_semantics=("parallel",)),
    )(z0, ga, be, m, v, wt)


def _pool(z, ga, be, m, v):
    c = z.shape[-1]

    def body(z_ref, ga_ref, be_ref, m_ref, v_ref, x_ref):
        zmax = jnp.max(z_ref[...], axis=1)
        x_ref[...] = _leaky(
            _bn_apply(zmax, ga_ref[0], be_ref[0], m_ref[0], v_ref[0]))

    return pl.pallas_call(
        body,
        grid=(BN // _TG,),
        in_specs=[
            pl.BlockSpec((_TG, K, c), lambda t: (t, 0, 0)),
            pl.BlockSpec((1, c), lambda t: (0, 0)),
            pl.BlockSpec((1, c), lambda t: (0, 0)),
            pl.BlockSpec((1, c), lambda t: (0, 0)),
            pl.BlockSpec((1, c), lambda t: (0, 0)),
        ],
        out_specs=pl.BlockSpec((_TG, c), lambda t: (t, 0)),
        out_shape=jax.ShapeDtypeStruct((BN, c), jnp.float32),
        compiler_params=pltpu.CompilerParams(
            dimension_semantics=("parallel",)),
    )(z, ga, be, m, v)


def _w5_stats(x1, x2, x3, w5t):
    th = 512
    nt = N // th

    def body(x1_ref, x2_ref, x3_ref, w_ref, st_ref, mx_ref):
        cat = jnp.concatenate([x1_ref[...], x2_ref[...], x3_ref[...]], axis=1)
        z5 = lax.dot_general(cat.astype(_BF), w_ref[...],
                             (((1,), (0,)), ((), ())),
                             preferred_element_type=jnp.float32)
        b = pl.program_id(0)
        t = pl.program_id(1)
        _stats_update(st_ref, z5, (b == 0) & (t == 0))
        tm = jnp.max(z5, axis=0)[None, None, :]

        @pl.when(t == 0)
        def _():
            mx_ref[...] = tm

        @pl.when(t > 0)
        def _():
            mx_ref[...] = jnp.maximum(mx_ref[...], tm)

    xspec = pl.BlockSpec((th, 64), lambda b, t: (b * nt + t, 0))
    return pl.pallas_call(
        body,
        grid=(B, nt),
        in_specs=[xspec, xspec, xspec,
                  pl.BlockSpec((192, 1024), lambda b, t: (0, 0))],
        out_specs=[
            pl.BlockSpec((8, 1024), lambda b, t: (0, 0)),
            pl.BlockSpec((1, 1, 1024), lambda b, t: (b, 0, 0)),
        ],
        out_shape=[
            jax.ShapeDtypeStruct((8, 1024), jnp.float32),
            jax.ShapeDtypeStruct((B, 1, 1024), jnp.float32),
        ],
        compiler_params=pltpu.CompilerParams(
            dimension_semantics=("arbitrary", "arbitrary")),
    )(x1, x2, x3, w5t)


def _head_small(zmax5p, g5, b5, m5, v5, labelp, w6t, g6, b6, p0g, p0l):
    def body(zm_ref, g5_ref, b5_ref, m5_ref, v5_ref, lab_ref, w6_ref,
             g6_ref, b6_ref, p0g_ref, p0l_ref, gl_ref):
        g_act = _leaky(_bn_apply(zm_ref[...], g5_ref[0], b5_ref[0],
                                 m5_ref[0], v5_ref[0]))
        z6 = lax.dot_general(lab_ref[...].astype(_BF), w6_ref[...],
                             (((1,), (0,)), ((), ())),
                             preferred_element_type=jnp.float32)
        row = lax.broadcasted_iota(jnp.int32, (8, 64), 0)
        mask = row < B
        m6 = jnp.sum(jnp.where(mask, z6, 0.0), axis=0) / B
        v6 = jnp.sum(jnp.where(mask, (z6 - m6[None, :]) ** 2, 0.0),
                     axis=0) / B
        l_act = _leaky(g6_ref[0] * (z6 - m6[None, :])
                       / jnp.sqrt(v6[None, :] + EPS) + b6_ref[0])
        gl = (lax.dot_general(g_act.astype(_BF), p0g_ref[...],
                              (((1,), (0,)), ((), ())),
                              preferred_element_type=jnp.float32)
              + lax.dot_general(l_act.astype(_BF), p0l_ref[...],
                                (((1,), (0,)), ((), ())),
                                preferred_element_type=jnp.float32))
        gl_ref[...] = gl

    full = lambda *s: pl.BlockSpec(s, lambda: tuple(0 for _ in s))
    return pl.pallas_call(
        body,
        in_specs=[
            full(8, 1024), full(1, 1024), full(1, 1024), full(1, 1024),
            full(1, 1024), full(8, 16), full(16, 64), full(1, 64),
            full(1, 64), full(1024, 256), full(64, 256),
        ],
        out_specs=full(8, 256),
        out_shape=jax.ShapeDtypeStruct((8, 256), jnp.float32),
    )(zmax5p, g5, b5, m5, v5, labelp, w6t, g6, b6, p0g, p0l)


def _p0(x1, x2, x3, p0xt, gl3):
    th = 512
    nt = N // th

    def body(x1_ref, x2_ref, x3_ref, w_ref, gl_ref, z_ref, st_ref):
        cat = jnp.concatenate([x1_ref[...], x2_ref[...], x3_ref[...]], axis=1)
        z = lax.dot_general(cat.astype(_BF), w_ref[...],
                            (((1,), (0,)), ((), ())),
                            preferred_element_type=jnp.float32)
        z = z + gl_ref[0]
        z_ref[...] = z
        b = pl.program_id(0)
        t = pl.program_id(1)
        _stats_update(st_ref, z, (b == 0) & (t == 0))

    xspec = pl.BlockSpec((th, 64), lambda b, t: (b * nt + t, 0))
    return pl.pallas_call(
        body,
        grid=(B, nt),
        in_specs=[xspec, xspec, xspec,
                  pl.BlockSpec((192, 256), lambda b, t: (0, 0)),
                  pl.BlockSpec((1, 1, 256), lambda b, t: (b, 0, 0))],
        out_specs=[
            pl.BlockSpec((th, 256), lambda b, t: (b * nt + t, 0)),
            pl.BlockSpec((8, 256), lambda b, t: (0, 0)),
        ],
        out_shape=[
            jax.ShapeDtypeStruct((BN, 256), jnp.float32),
            jax.ShapeDtypeStruct((8, 256), jnp.float32),
        ],
        compiler_params=pltpu.CompilerParams(
            dimension_semantics=("arbitrary", "arbitrary")),
    )(x1, x2, x3, p0xt, gl3)


def _p_mid(z_in, ga, be, m, v, wt, cout, want_stats):
    cin = z_in.shape[-1]
    th = 512

    def body(z_ref, ga_ref, be_ref, m_ref, v_ref, w_ref, z_out_ref,
             *maybe_st):
        f = _leaky(_bn_apply(z_ref[...], ga_ref[0], be_ref[0],
                             m_ref[0], v_ref[0]))
        z = lax.dot_general(f.astype(_BF), w_ref[...],
                            (((1,), (0,)), ((), ())),
                            preferred_element_type=jnp.float32)
        z_out_ref[...] = z
        if want_stats:
            _stats_update(maybe_st[0], z, pl.program_id(0) == 0)

    out_specs = [pl.BlockSpec((th, cout), lambda t: (t, 0))]
    out_shape = [jax.ShapeDtypeStruct((BN, cout), jnp.float32)]
    if want_stats:
        out_specs.append(pl.BlockSpec((8, cout), lambda t: (0, 0)))
        out_shape.append(jax.ShapeDtypeStruct((8, cout), jnp.float32))
    res = pl.pallas_call(
        body,
        grid=(BN // th,),
        in_specs=[
            pl.BlockSpec((th, cin), lambda t: (t, 0)),
            pl.BlockSpec((1, cin), lambda t: (0, 0)),
            pl.BlockSpec((1, cin), lambda t: (0, 0)),
            pl.BlockSpec((1, cin), lambda t: (0, 0)),
            pl.BlockSpec((1, cin), lambda t: (0, 0)),
            pl.BlockSpec((cin, cout), lambda t: (0, 0)),
        ],
        out_specs=out_specs,
        out_shape=out_shape,
        compiler_params=pltpu.CompilerParams(
            dimension_semantics=("arbitrary",)),
    )(z_in, ga, be, m, v, wt)
    return res if want_stats else (res[0], None)


def _edge_stage(x, d, w2d, ga0, be0, wt1, ga1, be1, xs=None):
    xtb = jnp.transpose(x, (0, 2, 1)).astype(_BF)
    idx = _knn(x, xtb, d)
    idxg = (jnp.arange(B, dtype=jnp.int32)[:, None, None] * N
            + jnp.transpose(idx[:, :K, :], (0, 2, 1))).reshape(ROWS)
    xflat = x.reshape(BN, d)
    gath = _sc_gather(xflat, idxg, d).reshape(BN, K, d)
    z0, st0 = _conv0(gath, xflat,
                     None if xs is None else xs.reshape(BN, d), w2d, d, 64)
    m0 = (jnp.sum(st0[:, 0, :], axis=0) * (1.0 / ROWS)).reshape(1, -1)
    v0 = _var_pass(z0, m0, float(ROWS))
    if wt1 is None:
        return _pool(z0, ga0, be0, m0, v0)
    z1, st1 = _conv_mid(z0, ga0, be0, m0, v0, wt1, 64)
    m1 = (jnp.sum(st1[:, 0, :], axis=0) * (1.0 / ROWS)).reshape(1, -1)
    v1 = _var_pass(z1, m1, float(ROWS))
    return _pool(z1, ga1, be1, m1, v1)


def kernel(points, label, params):
    p = params
    r1 = lambda a: a.reshape(1, -1)

    x0r = jnp.transpose(points, (0, 2, 1))
    x0 = jnp.pad(x0r, ((0, 0), (0, 0), (0, 13)))
    xs0 = jnp.pad(x0r, ((0, 0), (0, 0), (3, 10)))
    w0 = jnp.pad(p['W0'].T, ((0, 10), (0, 0)))
    x1 = _edge_stage(x0, 16, w0.astype(_BF), r1(p['g0']), r1(p['b0']),
                     p['W1'].T.astype(_BF), r1(p['g1']), r1(p['b1']),
                     xs=xs0)

    w2 = jnp.concatenate([p['W2'][:, :64].T, p['W2'][:, 64:].T], axis=0)
    x2 = _edge_stage(x1.reshape(B, N, 64), 64, w2.astype(_BF),
                     r1(p['g2']), r1(p['b2']),
                     p['W3'].T.astype(_BF), r1(p['g3']), r1(p['b3']))

    w4 = jnp.concatenate([p['W4'][:, :64].T, p['W4'][:, 64:].T], axis=0)
    x3 = _edge_stage(x2.reshape(B, N, 64), 64, w4.astype(_BF),
                     r1(p['g4']), r1(p['b4']), None, None, None)

    st5, zmax5 = _w5_stats(x1, x2, x3, p['W5'].T.astype(_BF))
    m5, v5 = _finalize_stats(st5, float(BN))
    zmax5p = jnp.pad(zmax5.reshape(B, 1024), ((0, 8 - B), (0, 0)))
    labelp = jnp.pad(label[:, :, 0], ((0, 8 - B), (0, 0)))
    gl = _head_small(zmax5p, r1(p['g5']), r1(p['b5']), m5, v5, labelp,
                     p['W6'].T.astype(_BF), r1(p['g6']), r1(p['b6']),
                     p['P0'][:, 192:1216].T.astype(_BF),
                     p['P0'][:, 1216:1280].T.astype(_BF))
    gl3 = gl[:B].reshape(B, 1, 256)

    zp0, stp0 = _p0(x1, x2, x3, p['P0'][:, :192].T.astype(_BF), gl3)
    mp0, vp0 = _finalize_stats(stp0, float(BN))
    zp1, stp1 = _p_mid(zp0, r1(p['pg0']), r1(p['pb0']), mp0, vp0,
                       p['P1'].T.astype(_BF), 256, True)
    mp1, vp1 = _finalize_stats(stp1, float(BN))
    zp2, stp2 = _p_mid(zp1, r1(p['pg1']), r1(p['pb1']), mp1, vp1,
                       p['P2'].T.astype(_BF), 128, True)
    mp2, vp2 = _finalize_stats(stp2, float(BN))
    p3t = jnp.pad(p['P3'].T, ((0, 0), (0, 2)))
    out8, _ = _p_mid(zp2, r1(p['pg2']), r1(p['pb2']), mp2, vp2,
                     p3t.astype(_BF), 8, False)
    return jnp.transpose(out8.reshape(B, N, 8)[:, :, :6], (0, 2, 1))
